# baseline (speedup 1.0000x reference)
import functools

import jax
import jax.numpy as jnp
from jax import lax
from jax.experimental import pallas as pl
from jax.experimental.pallas import tpu as pltpu
from jax.experimental.pallas import tpu_sc as plsc

_NC, _NS = 2, 16
_NW = _NC * _NS
_BUCKET_BITS = 15
_WIN = 128
_WIN_BITS = 7
_DEPTH = 8
_SUB = 4
_RING = 64
_POS_BITS = 14


def _make_kernel(B, V, D):
  Vmain = (V // 128) * 128
  nwin_max = (1 << _BUCKET_BITS) >> _WIN_BITS
  mesh = plsc.VectorSubcoreMesh(core_axis_name="c", subcore_axis_name="s")

  @functools.partial(
      pl.kernel,
      mesh=mesh,
      out_type=jax.ShapeDtypeStruct((B * D,), jnp.float32),
      compiler_params=pltpu.CompilerParams(
          needs_layout_passes=False, disable_bounds_checks=True),
      scratch_types=[
          pltpu.VMEM((4096,), jnp.int32),
          pltpu.VMEM((B + 16,), jnp.int32),
          pltpu.VMEM((B + 16,), jnp.int32),
          pltpu.VMEM((nwin_max + 16,), jnp.int32),
          pltpu.VMEM((nwin_max + 16,), jnp.int32),
          pltpu.VMEM((nwin_max + 16,), jnp.int32),
          pltpu.VMEM((_DEPTH, D // 8, 8, _WIN), jnp.float32),
          pltpu.VMEM((D, 128), jnp.float32),
          pltpu.VMEM((_RING, D), jnp.float32),
          pltpu.SemaphoreType.DMA,
          pltpu.SemaphoreType.DMA,
      ],
  )
  def k(idx_hbm, tabT_hbm, out_hbm,
        idx_c, plist, slist, whist, wstart, wcur, panel, tailp, stage,
        sem_p, sem_o):
    wid = lax.axis_index("s") * _NC + lax.axis_index("c")
    base = wid << _BUCKET_BITS
    iota = lax.iota(jnp.int32, 16)
    zeros = jnp.zeros((16,), jnp.int32)
    lane0 = iota == 0

    def bucket_chunk(chunk, cnt0):
      def bucket_body(g, cnt):
        vg = idx_c[pl.ds(g * 16, 16)]
        m = (vg >> _BUCKET_BITS) == wid
        mi = m.astype(jnp.int32)
        ex = plsc.cumsum(mi) - mi
        slots = ex + cnt
        pos = chunk * 4096 + g * 16 + iota
        packed = ((vg - base) << _POS_BITS) | pos
        plsc.store_scatter(plist, [slots], packed, mask=m)
        return cnt + jnp.sum(mi)
      return lax.fori_loop(0, 4096 // 16, bucket_body, cnt0)

    cnt = jnp.int32(0)
    for chunk in range(B // 4096):
      pltpu.sync_copy(idx_hbm.at[pl.ds(chunk * 4096, 4096)], idx_c)
      cnt = bucket_chunk(chunk, cnt)

    def lane_extract(vec, l):
      return jnp.sum(jnp.where(iota == l, vec, 0))

    def list_entry(ref, j):
      vg = ref[pl.ds((j >> 4) * 16, 16)]
      return lane_extract(vg, j & 15)

    for z in range((nwin_max + 16) // 16):
      whist[pl.ds(z * 16, 16)] = zeros

    def hist_body(j, _):
      pk = list_entry(plist, j)
      w_spl = lax.broadcast(pk >> (_POS_BITS + _WIN_BITS), (16,))
      c = plsc.load_gather(whist, [w_spl])
      plsc.store_scatter(whist, [w_spl], c + 1, mask=lane0)
      return 0
    lax.fori_loop(0, cnt, hist_body, 0)

    def pfx_body(z, run):
      h = whist[pl.ds(z * 16, 16)]
      ex = plsc.cumsum(h) - h + run
      wstart[pl.ds(z * 16, 16)] = ex
      wcur[pl.ds(z * 16, 16)] = ex
      return run + jnp.sum(h)
    lax.fori_loop(0, (nwin_max + 16) // 16, pfx_body, jnp.int32(0))

    def place_body(j, _):
      pk = list_entry(plist, j)
      w_spl = lax.broadcast(pk >> (_POS_BITS + _WIN_BITS), (16,))
      s = plsc.load_gather(wcur, [w_spl])
      plsc.store_scatter(slist, [s], lax.broadcast(pk, (16,)), mask=lane0)
      plsc.store_scatter(wcur, [w_spl], s + 1, mask=lane0)
      return 0
    lax.fori_loop(0, cnt, place_body, 0)

    def win_range(w):
      w_spl = lax.broadcast(w, (16,))
      st = lane_extract(plsc.load_gather(wstart, [w_spl]), 0)
      ct = lane_extract(plsc.load_gather(whist, [w_spl]), 0)
      return st, ct

    def extract_entry(j, outcnt, waits, from_tail, buf=None):
      pk = list_entry(slist, j)
      vloc = pk >> _POS_BITS
      p_s = pk & ((1 << _POS_BITS) - 1)
      slot = outcnt & (_RING - 1)
      if from_tail:
        c_spl = lax.broadcast(vloc - (Vmain - base), (16,))
        for kk in range(D // 16):
          val = plsc.load_gather(tailp, [iota + 16 * kk, c_spl])
          stage[slot, pl.ds(kk * 16, 16)] = val
      else:
        v_spl = lax.broadcast(vloc & (_WIN - 1), (16,))
        b_spl = lax.broadcast(buf, (16,))
        for kk in range(D // 16):
          e = iota + 16 * kk
          val = plsc.load_gather(
              panel, [b_spl, e >> 3, e & 7, v_spl])
          stage[slot, pl.ds(kk * 16, 16)] = val
      pltpu.async_copy(stage.at[slot], out_hbm.at[pl.ds(p_s * D, D)], sem_o)
      outcnt = outcnt + 1
      do_wait = outcnt > _RING

      @pl.when(do_wait)
      def _():
        pltpu.make_async_copy(
            out_hbm.at[pl.ds(0, D)], stage.at[0], sem_o).wait()

      waits = waits + jnp.where(do_wait, 1, 0).astype(jnp.int32)
      return outcnt, waits

    rem = jnp.maximum(Vmain - base, 0)
    nwin = jnp.minimum(rem >> _WIN_BITS, nwin_max)

    def fire(win):
      off = pl.multiple_of(base + win * _WIN, _WIN)
      buf = win & (_DEPTH - 1)
      for t in range(D // 8):
        pltpu.async_copy(
            tabT_hbm.at[pl.ds(t * 8, 8), pl.ds(off, _WIN)],
            panel.at[buf, t], sem_p)

    for pre in range(_DEPTH - 1):
      @pl.when(nwin > pre)
      def _(pre=pre):
        fire(pre)

    def win_body(win, carry):
      @pl.when(win + (_DEPTH - 1) < nwin)
      def _():
        fire(win + (_DEPTH - 1))

      buf = win & (_DEPTH - 1)
      for t in range(D // 8):
        pltpu.make_async_copy(
            tabT_hbm.at[pl.ds(t * 8, 8), pl.ds(0, _WIN)],
            panel.at[buf, t], sem_p).wait()
      st, ct = win_range(win)

      def ex_body(t, c):
        return extract_entry(st + t, c[0], c[1], False, buf)

      return lax.fori_loop(0, ct, ex_body, carry)

    carry = lax.fori_loop(0, nwin, win_body,
                          (jnp.int32(0), jnp.int32(0)))

    is_tail_owner = (Vmain >> _BUCKET_BITS) == wid

    @pl.when(is_tail_owner)
    def _():
      off_tail = pl.multiple_of(Vmain + wid * 0, 128)
      pltpu.sync_copy(tabT_hbm.at[:, pl.ds(off_tail, 128)], tailp)

    st_t, ct_t = win_range(nwin)

    def tail_body(t, c):
      return extract_entry(st_t + t, c[0], c[1], True)

    carry = lax.fori_loop(0, ct_t, tail_body, carry)

    outcnt, waits = carry

    def drain(_, x):
      pltpu.make_async_copy(
          out_hbm.at[pl.ds(0, D)], stage.at[0], sem_o).wait()
      return x

    lax.fori_loop(0, outcnt - waits, drain, 0)

  return k


def kernel(input, table):
  B = input.shape[0]
  V, D = table.shape
  assert D == 64 and B % 4096 == 0 and B <= (1 << _POS_BITS)
  tabT = table.T
  out = _make_kernel(B, V, D)(input, tabT)
  return out.reshape((1, -1))


# speedup vs baseline: 1.0057x; 1.0057x over previous
import functools

import jax
import jax.numpy as jnp
from jax import lax
from jax.experimental import pallas as pl
from jax.experimental.pallas import tpu as pltpu
from jax.experimental.pallas import tpu_sc as plsc

_NC, _NS = 2, 16
_NW = _NC * _NS
_BUCKET_BITS = 15
_WIN = 256
_WIN_BITS = 8
_DEPTH = 4
_SUB = 4
_RING = 64
_POS_BITS = 14


def _make_kernel(B, V, D):
  Vmain = (V // 128) * 128
  nwin_max = (1 << _BUCKET_BITS) >> _WIN_BITS
  mesh = plsc.VectorSubcoreMesh(core_axis_name="c", subcore_axis_name="s")

  @functools.partial(
      pl.kernel,
      mesh=mesh,
      out_type=jax.ShapeDtypeStruct((B * D,), jnp.float32),
      compiler_params=pltpu.CompilerParams(
          needs_layout_passes=False, disable_bounds_checks=True),
      scratch_types=[
          pltpu.VMEM((4096,), jnp.int32),
          pltpu.VMEM((B + 16,), jnp.int32),
          pltpu.VMEM((B + 16,), jnp.int32),
          pltpu.VMEM((nwin_max + 16,), jnp.int32),
          pltpu.VMEM((nwin_max + 16,), jnp.int32),
          pltpu.VMEM((nwin_max + 16,), jnp.int32),
          pltpu.VMEM((_DEPTH, D, _WIN), jnp.float32),
          pltpu.VMEM((D, 128), jnp.float32),
          pltpu.VMEM((_RING, D), jnp.float32),
          pltpu.SemaphoreType.DMA,
          pltpu.SemaphoreType.DMA,
      ],
  )
  def k(idx_hbm, tabT_hbm, out_hbm,
        idx_c, plist, slist, whist, wstart, wcur, panel, tailp, stage,
        sem_p, sem_o):
    wid = lax.axis_index("s") * _NC + lax.axis_index("c")
    base = wid << _BUCKET_BITS
    iota = lax.iota(jnp.int32, 16)
    zeros = jnp.zeros((16,), jnp.int32)
    lane0 = iota == 0

    def bucket_chunk(chunk, cnt0):
      def bucket_body(g, cnt):
        vg = idx_c[pl.ds(g * 16, 16)]
        m = (vg >> _BUCKET_BITS) == wid
        mi = m.astype(jnp.int32)
        ex = plsc.cumsum(mi) - mi
        slots = ex + cnt
        pos = chunk * 4096 + g * 16 + iota
        packed = ((vg - base) << _POS_BITS) | pos
        plsc.store_scatter(plist, [slots], packed, mask=m)
        return cnt + jnp.sum(mi)
      return lax.fori_loop(0, 4096 // 16, bucket_body, cnt0)

    cnt = jnp.int32(0)
    for chunk in range(B // 4096):
      pltpu.sync_copy(idx_hbm.at[pl.ds(chunk * 4096, 4096)], idx_c)
      cnt = bucket_chunk(chunk, cnt)

    def lane_extract(vec, l):
      return jnp.sum(jnp.where(iota == l, vec, 0))

    def list_entry(ref, j):
      vg = ref[pl.ds((j >> 4) * 16, 16)]
      return lane_extract(vg, j & 15)

    for z in range((nwin_max + 16) // 16):
      whist[pl.ds(z * 16, 16)] = zeros

    def hist_body(j, _):
      pk = list_entry(plist, j)
      w_spl = lax.broadcast(pk >> (_POS_BITS + _WIN_BITS), (16,))
      c = plsc.load_gather(whist, [w_spl])
      plsc.store_scatter(whist, [w_spl], c + 1, mask=lane0)
      return 0
    lax.fori_loop(0, cnt, hist_body, 0)

    def pfx_body(z, run):
      h = whist[pl.ds(z * 16, 16)]
      ex = plsc.cumsum(h) - h + run
      wstart[pl.ds(z * 16, 16)] = ex
      wcur[pl.ds(z * 16, 16)] = ex
      return run + jnp.sum(h)
    lax.fori_loop(0, (nwin_max + 16) // 16, pfx_body, jnp.int32(0))

    def place_body(j, _):
      pk = list_entry(plist, j)
      w_spl = lax.broadcast(pk >> (_POS_BITS + _WIN_BITS), (16,))
      s = plsc.load_gather(wcur, [w_spl])
      plsc.store_scatter(slist, [s], lax.broadcast(pk, (16,)), mask=lane0)
      plsc.store_scatter(wcur, [w_spl], s + 1, mask=lane0)
      return 0
    lax.fori_loop(0, cnt, place_body, 0)

    def win_range(w):
      w_spl = lax.broadcast(w, (16,))
      st = lane_extract(plsc.load_gather(wstart, [w_spl]), 0)
      ct = lane_extract(plsc.load_gather(whist, [w_spl]), 0)
      return st, ct

    def extract_entry(j, outcnt, waits, from_tail, buf=None):
      pk = list_entry(slist, j)
      vloc = pk >> _POS_BITS
      p_s = pk & ((1 << _POS_BITS) - 1)
      slot = outcnt & (_RING - 1)
      if from_tail:
        c_spl = lax.broadcast(vloc - (Vmain - base), (16,))
        for kk in range(D // 16):
          val = plsc.load_gather(tailp, [iota + 16 * kk, c_spl])
          stage[slot, pl.ds(kk * 16, 16)] = val
      else:
        v_spl = lax.broadcast(vloc & (_WIN - 1), (16,))
        b_spl = lax.broadcast(buf, (16,))
        for kk in range(D // 16):
          val = plsc.load_gather(
              panel, [b_spl, iota + 16 * kk, v_spl])
          stage[slot, pl.ds(kk * 16, 16)] = val
      pltpu.async_copy(stage.at[slot], out_hbm.at[pl.ds(p_s * D, D)], sem_o)
      outcnt = outcnt + 1
      do_wait = outcnt > _RING

      @pl.when(do_wait)
      def _():
        pltpu.make_async_copy(
            out_hbm.at[pl.ds(0, D)], stage.at[0], sem_o).wait()

      waits = waits + jnp.where(do_wait, 1, 0).astype(jnp.int32)
      return outcnt, waits

    rem = jnp.maximum(Vmain - base, 0)
    nwin = jnp.minimum(rem >> _WIN_BITS, nwin_max)

    def fire(win):
      off = pl.multiple_of(base + win * _WIN, _WIN)
      buf = win & (_DEPTH - 1)
      pltpu.async_copy(
          tabT_hbm.at[:, pl.ds(off, _WIN)], panel.at[buf], sem_p)

    for pre in range(_DEPTH - 1):
      @pl.when(nwin > pre)
      def _(pre=pre):
        fire(pre)

    def win_body(win, carry):
      @pl.when(win + (_DEPTH - 1) < nwin)
      def _():
        fire(win + (_DEPTH - 1))

      buf = win & (_DEPTH - 1)
      pltpu.make_async_copy(
          tabT_hbm.at[:, pl.ds(0, _WIN)], panel.at[buf], sem_p).wait()
      st, ct = win_range(win)

      def ex_body(t, c):
        return extract_entry(st + t, c[0], c[1], False, buf)

      return lax.fori_loop(0, ct, ex_body, carry)

    carry = lax.fori_loop(0, nwin, win_body,
                          (jnp.int32(0), jnp.int32(0)))

    is_tail_owner = (Vmain >> _BUCKET_BITS) == wid

    @pl.when(is_tail_owner)
    def _():
      off_tail = pl.multiple_of(Vmain + wid * 0, 128)
      pltpu.sync_copy(tabT_hbm.at[:, pl.ds(off_tail, 128)], tailp)

    st_t, ct_t = win_range(nwin)

    def tail_body(t, c):
      return extract_entry(st_t + t, c[0], c[1], True)

    carry = lax.fori_loop(0, ct_t, tail_body, carry)

    outcnt, waits = carry

    def drain(_, x):
      pltpu.make_async_copy(
          out_hbm.at[pl.ds(0, D)], stage.at[0], sem_o).wait()
      return x

    lax.fori_loop(0, outcnt - waits, drain, 0)

  return k


def kernel(input, table):
  B = input.shape[0]
  V, D = table.shape
  assert D == 64 and B % 4096 == 0 and B <= (1 << _POS_BITS)
  tabT = table.T
  out = _make_kernel(B, V, D)(input, tabT)
  return out.reshape((1, -1))
